# Initial kernel scaffold; baseline (speedup 1.0000x reference)
import jax
import jax.numpy as jnp
from jax.experimental import pallas as pl


def kernel(node_w, node_b, graph_token, masif_w1, masif_b1, masif_w2, masif_b2, gb_w1, gb_b1, gb_w2, gb_b2, fuse_w, fuse_b, node_feat, masif_desc_straight, gbscore):
    raise NotImplementedError("write your pallas kernel here")


# speedup vs baseline: 1.8418x; 1.8418x over previous
import math

import jax
import jax.numpy as jnp
from jax import lax
from jax.experimental import pallas as pl
from jax.experimental.pallas import tpu as pltpu

_H = 768


def _gelu(x):
    return 0.5 * x * (1.0 + lax.erf(x * (1.0 / math.sqrt(2.0))))


def _fused_block(node_ref, masif_ref, gbs_ref,
                 mw1_ref, mb1_ref, mwf_ref,
                 gw1_ref, gb1_ref, gwf_ref,
                 tokb_ref, nw_ref, nb_ref, o_ref):
    gb, n, f = node_ref.shape
    h = _H

    msum = jnp.sum(masif_ref[...], axis=1)
    mh = _gelu(jnp.dot(msum.astype(jnp.bfloat16), mw1_ref[...],
                       preferred_element_type=jnp.float32) + mb1_ref[...])
    gh = _gelu(jnp.dot(gbs_ref[...].astype(jnp.bfloat16), gw1_ref[...],
                       preferred_element_type=jnp.float32) + gb1_ref[...])
    tok = (jnp.dot(mh.astype(jnp.bfloat16), mwf_ref[...],
                   preferred_element_type=jnp.float32)
           + jnp.dot(gh.astype(jnp.bfloat16), gwf_ref[...],
                     preferred_element_type=jnp.float32)
           + tokb_ref[...])

    nodes = jnp.dot(node_ref[...].reshape(gb * n, f).astype(jnp.bfloat16),
                    nw_ref[...], preferred_element_type=jnp.float32)
    nodes = nodes.reshape(gb, n, h) + nb_ref[...].reshape(1, 1, h)

    o_ref[...] = jnp.concatenate([tok.reshape(gb, 1, h), nodes], axis=1)


def _resident(shape):
    nd = len(shape)
    return pl.BlockSpec(shape, lambda i: (0,) * nd)


def kernel(node_w, node_b, graph_token,
           masif_w1, masif_b1, masif_w2, masif_b2,
           gb_w1, gb_b1, gb_w2, gb_b2,
           fuse_w, fuse_b,
           node_feat, masif_desc_straight, gbscore):
    h = _H
    g, n, f = node_feat.shape
    m = masif_desc_straight.shape[1]
    dg = gbscore.shape[1]

    gb = min(g, 64)
    while g % gb:
        gb -= 1

    fwm = fuse_w[h:2 * h]
    fwg = fuse_w[2 * h:3 * h]
    tok_bias = (graph_token @ fuse_w[:h] + fuse_b
                + masif_b2 @ fwm + gb_b2 @ fwg)
    mwf = (masif_w2 @ fwm).astype(jnp.bfloat16)
    gwf = (gb_w2 @ fwg).astype(jnp.bfloat16)
    mw1s = (masif_w1 * (1.0 / m)).astype(jnp.bfloat16)
    gw1b = gb_w1.astype(jnp.bfloat16)
    nwb = node_w.astype(jnp.bfloat16)

    w_args = (mw1s, masif_b1, mwf, gw1b, gb_b1, gwf, tok_bias, nwb, node_b)
    w_specs = [_resident(tuple(w.shape)) for w in w_args]

    return pl.pallas_call(
        _fused_block,
        out_shape=jax.ShapeDtypeStruct((g, n + 1, h), jnp.float32),
        grid=(g // gb,),
        in_specs=[
            pl.BlockSpec((gb, n, f), lambda i: (i, 0, 0)),
            pl.BlockSpec((gb, m, masif_desc_straight.shape[2]),
                         lambda i: (i, 0, 0)),
            pl.BlockSpec((gb, dg), lambda i: (i, 0)),
        ] + w_specs,
        out_specs=pl.BlockSpec((gb, n + 1, h), lambda i: (i, 0, 0)),
        compiler_params=pltpu.CompilerParams(
            dimension_semantics=("parallel",),
            vmem_limit_bytes=(64 << 20) * 4 // 5),
    )(node_feat, masif_desc_straight, gbscore, *w_args)


# speedup vs baseline: 1.8861x; 1.0241x over previous
import math

import jax
import jax.numpy as jnp
from jax import lax
from jax.experimental import pallas as pl
from jax.experimental.pallas import tpu as pltpu

_H = 768


def _gelu(x):
    return 0.5 * x * (1.0 + lax.erf(x * (1.0 / math.sqrt(2.0))))


def _fused_block(node_ref, masif_ref, gbs_ref,
                 mw1_ref, mb1_ref, mwf_ref,
                 gw1_ref, gb1_ref, gwf_ref,
                 tokb_ref, nw_ref, nb_ref, o_ref):
    gb, n, f = node_ref.shape
    h = _H

    mh = _gelu(jnp.dot(masif_ref[...].astype(jnp.bfloat16), mw1_ref[...],
                       preferred_element_type=jnp.float32) + mb1_ref[...])
    gh = _gelu(jnp.dot(gbs_ref[...].astype(jnp.bfloat16), gw1_ref[...],
                       preferred_element_type=jnp.float32) + gb1_ref[...])
    tok = (jnp.dot(mh.astype(jnp.bfloat16), mwf_ref[...],
                   preferred_element_type=jnp.float32)
           + jnp.dot(gh.astype(jnp.bfloat16), gwf_ref[...],
                     preferred_element_type=jnp.float32)
           + tokb_ref[...])

    nodes = jnp.dot(node_ref[...].reshape(gb * n, f).astype(jnp.bfloat16),
                    nw_ref[...], preferred_element_type=jnp.float32)
    nodes = nodes.reshape(gb, n, h) + nb_ref[...].reshape(1, 1, h)

    o_ref[...] = jnp.concatenate([tok.reshape(gb, 1, h), nodes], axis=1)


def _resident(shape):
    nd = len(shape)
    return pl.BlockSpec(shape, lambda i: (0,) * nd)


def kernel(node_w, node_b, graph_token,
           masif_w1, masif_b1, masif_w2, masif_b2,
           gb_w1, gb_b1, gb_w2, gb_b2,
           fuse_w, fuse_b,
           node_feat, masif_desc_straight, gbscore):
    h = _H
    g, n, f = node_feat.shape
    m = masif_desc_straight.shape[1]
    dg = gbscore.shape[1]

    gb = min(g, 64)
    while g % gb:
        gb -= 1

    fwm = fuse_w[h:2 * h]
    fwg = fuse_w[2 * h:3 * h]
    tok_bias = (graph_token @ fuse_w[:h] + fuse_b
                + masif_b2 @ fwm + gb_b2 @ fwg)
    mwf = (masif_w2 @ fwm).astype(jnp.bfloat16)
    gwf = (gb_w2 @ fwg).astype(jnp.bfloat16)
    mw1s = jnp.tile(masif_w1 * (1.0 / m), (m, 1)).astype(jnp.bfloat16)
    gw1b = gb_w1.astype(jnp.bfloat16)
    nwb = node_w.astype(jnp.bfloat16)

    dmf = m * masif_desc_straight.shape[2]
    masif_flat = masif_desc_straight.reshape(g, dmf)

    w_args = (mw1s, masif_b1, mwf, gw1b, gb_b1, gwf, tok_bias, nwb, node_b)
    w_specs = [_resident(tuple(w.shape)) for w in w_args]

    return pl.pallas_call(
        _fused_block,
        out_shape=jax.ShapeDtypeStruct((g, n + 1, h), jnp.float32),
        grid=(g // gb,),
        in_specs=[
            pl.BlockSpec((gb, n, f), lambda i: (i, 0, 0)),
            pl.BlockSpec((gb, dmf), lambda i: (i, 0)),
            pl.BlockSpec((gb, dg), lambda i: (i, 0)),
        ] + w_specs,
        out_specs=pl.BlockSpec((gb, n + 1, h), lambda i: (i, 0, 0)),
        compiler_params=pltpu.CompilerParams(
            dimension_semantics=("parallel",),
            vmem_limit_bytes=(64 << 20) * 4 // 5),
    )(node_feat, masif_flat, gbscore, *w_args)
